# Initial kernel scaffold; baseline (speedup 1.0000x reference)
import functools

import jax
import jax.numpy as jnp
from jax import lax
from jax.experimental import pallas as pl
from jax.experimental.pallas import tpu as pltpu
from jax.experimental.pallas import tpu_sc as plsc

N = 10000
E = 320000
D = 128
T = 4
R = 16

NC = 2
NS = 16
NW = NC * NS
L = 16

CH = 128
NCH = E // CH
LANE_IT = CH // L


def _chunk_range(worker, num_workers):
    q = NCH // num_workers
    r = NCH % num_workers
    return worker * q + jnp.minimum(worker, r)


def _weights_body(src_hbm, tgt_hbm, nt_hbm, c1_hbm, c2_hbm, w1_hbm, w2_hbm,
                  nt_v, srcv, tgtv, segv, relv, onesv, cntv, w1v, w2v,
                  c1v, c2v, zbuf, cnt_sh, gsem):
    cid = lax.axis_index("c")
    sid = lax.axis_index("s")
    wid = cid * NS + sid

    pltpu.sync_copy(nt_hbm, nt_v)
    pltpu.sync_copy(c1_hbm, c1v)
    pltpu.sync_copy(c2_hbm, c2v)

    zeros16 = jnp.zeros((L,), jnp.float32)
    ones16 = jnp.ones((L,), jnp.float32)

    def init_bufs(i, carry):
        zbuf[pl.ds(i * L, L)] = zeros16
        return carry
    lax.fori_loop(0, (N * R // NS) // L, init_bufs, 0)
    for j in range(LANE_IT):
        onesv[pl.ds(j * L, L)] = ones16

    pltpu.sync_copy(zbuf, cnt_sh.at[pl.ds(sid * (N * R // NS), N * R // NS)])
    plsc.subcore_barrier()

    def compute_seg(base):
        pltpu.sync_copy(src_hbm.at[pl.ds(base, CH)], srcv)
        pltpu.sync_copy(tgt_hbm.at[pl.ds(base, CH)], tgtv)
        for k in range(LANE_IT):
            s16 = srcv[pl.ds(k * L, L)]
            t16 = tgtv[pl.ds(k * L, L)]
            ts = plsc.load_gather(nt_v, [s16])
            tt = plsc.load_gather(nt_v, [t16])
            rel = tt * T + ts
            segv[pl.ds(k * L, L)] = t16 * R + rel
            relv[pl.ds(k * L, L)] = rel

    c0 = _chunk_range(sid, NS)
    cn = NCH // NS + (sid < NCH % NS).astype(jnp.int32)

    def count_chunk(ci, carry):
        compute_seg((c0 + ci) * CH)
        pltpu.sync_copy(onesv, cnt_sh.at[segv], add=True)
        return carry
    lax.fori_loop(0, cn, count_chunk, 0)
    plsc.subcore_barrier()

    w0 = _chunk_range(wid, NW)
    wn = NCH // NW + (wid < NCH % NW).astype(jnp.int32)

    def w_chunk(ci, carry):
        base = (w0 + ci) * CH
        compute_seg(base)
        pltpu.async_copy(cnt_sh.at[segv], cntv, gsem).wait()
        for k in range(LANE_IT):
            c16 = cntv[pl.ds(k * L, L)]
            inv = 1.0 / jnp.maximum(c16, 1.0)
            rel16 = relv[pl.ds(k * L, L)]
            w1v[pl.ds(k * L, L)] = plsc.load_gather(c1v, [rel16]) * inv
            w2v[pl.ds(k * L, L)] = plsc.load_gather(c2v, [rel16]) * inv
        pltpu.sync_copy(w1v, w1_hbm.at[pl.ds(base, CH)])
        pltpu.sync_copy(w2v, w2_hbm.at[pl.ds(base, CH)])
        return carry
    lax.fori_loop(0, wn, w_chunk, 0)


def _weights(src, tgt, node_type, comp1, comp2):
    mesh = plsc.VectorSubcoreMesh(core_axis_name="c", subcore_axis_name="s")
    f = pl.kernel(
        _weights_body,
        out_type=(jax.ShapeDtypeStruct((E,), jnp.float32),
                  jax.ShapeDtypeStruct((E,), jnp.float32)),
        mesh=mesh,
        scratch_types=[
            pltpu.VMEM((N,), jnp.int32),
            pltpu.VMEM((CH,), jnp.int32),
            pltpu.VMEM((CH,), jnp.int32),
            pltpu.VMEM((CH,), jnp.int32),
            pltpu.VMEM((CH,), jnp.int32),
            pltpu.VMEM((CH,), jnp.float32),
            pltpu.VMEM((CH,), jnp.float32),
            pltpu.VMEM((CH,), jnp.float32),
            pltpu.VMEM((CH,), jnp.float32),
            pltpu.VMEM((R,), jnp.float32),
            pltpu.VMEM((R,), jnp.float32),
            pltpu.VMEM((N * R // NS,), jnp.float32),
            pltpu.VMEM_SHARED((N * R,), jnp.float32),
            pltpu.SemaphoreType.DMA,
        ],
        name="rgcn_weights",
    )
    return f(src, tgt, node_type, comp1, comp2)


def _agg_body(x_hbm, src_hbm, tgt_hbm, w_hbm, out_hbm,
              srcv, tgtv, wv, rows, agg_sh, gsem):
    cid = lax.axis_index("c")
    sid = lax.axis_index("s")
    wid = cid * NS + sid

    zeros16 = jnp.zeros((L,), jnp.float32)

    def zrow(i, carry):
        for j in range(D // L):
            rows[i, pl.ds(j * L, L)] = zeros16
        return carry
    lax.fori_loop(0, CH, zrow, 0)

    rows_per_tile = N // NS
    r0 = sid * rows_per_tile
    nrow = rows_per_tile // 5
    for i in range(5):
        pltpu.sync_copy(rows.at[pl.ds(0, nrow)],
                        agg_sh.at[pl.ds(r0 + i * nrow, nrow)])
    plsc.subcore_barrier()

    c0 = _chunk_range(wid, NW)
    cn = NCH // NW + (wid < NCH % NW).astype(jnp.int32)

    def chunk(ci, carry):
        base = (c0 + ci) * CH
        pltpu.sync_copy(src_hbm.at[pl.ds(base, CH)], srcv)
        pltpu.sync_copy(tgt_hbm.at[pl.ds(base, CH)], tgtv)
        pltpu.sync_copy(w_hbm.at[pl.ds(base, CH)], wv)
        pltpu.async_copy(x_hbm.at[srcv], rows, gsem).wait()

        def scale(i, icarry):
            w16 = plsc.load_gather(wv, [jnp.full((L,), i, jnp.int32)])
            for j in range(D // L):
                rows[i, pl.ds(j * L, L)] = rows[i, pl.ds(j * L, L)] * w16
            return icarry
        lax.fori_loop(0, CH, scale, 0)

        pltpu.sync_copy(rows, agg_sh.at[tgtv], add=True)
        return carry
    lax.fori_loop(0, cn, chunk, 0)
    plsc.subcore_barrier()

    for i in range(5):
        pltpu.sync_copy(agg_sh.at[pl.ds(r0 + i * nrow, nrow)],
                        out_hbm.at[cid, pl.ds(r0 + i * nrow, nrow)])


def _agg(x, src, tgt, w):
    mesh = plsc.VectorSubcoreMesh(core_axis_name="c", subcore_axis_name="s")
    f = pl.kernel(
        _agg_body,
        out_type=jax.ShapeDtypeStruct((NC, N, D), jnp.float32),
        mesh=mesh,
        scratch_types=[
            pltpu.VMEM((CH,), jnp.int32),
            pltpu.VMEM((CH,), jnp.int32),
            pltpu.VMEM((CH,), jnp.float32),
            pltpu.VMEM((CH, D), jnp.float32),
            pltpu.VMEM_SHARED((N, D), jnp.float32),
            pltpu.SemaphoreType.DMA,
        ],
        name="rgcn_agg",
    )
    return f(x, src, tgt, w)


_BR = 1000


def _tc_body(a0_ref, a1_ref, x_ref, v_ref, root_ref, b_ref, o_ref):
    agg = a0_ref[...] + a1_ref[...]
    acc = jnp.dot(agg, v_ref[...], preferred_element_type=jnp.float32)
    acc += jnp.dot(x_ref[...], root_ref[...], preferred_element_type=jnp.float32)
    o_ref[...] = jax.nn.sigmoid(acc + b_ref[...])


def _tc(agg0, agg1, x, V, root, bias):
    grid = (N // _BR,)
    row_spec = pl.BlockSpec((_BR, D), lambda i: (i, 0))
    mat_spec = pl.BlockSpec((D, D), lambda i: (0, 0))
    return pl.pallas_call(
        _tc_body,
        grid=grid,
        in_specs=[row_spec, row_spec, row_spec, mat_spec, mat_spec,
                  pl.BlockSpec((1, D), lambda i: (0, 0))],
        out_specs=row_spec,
        out_shape=jax.ShapeDtypeStruct((N, D), jnp.float32),
    )(agg0, agg1, x, V, root, bias)


def kernel(x, edge_index, node_type, V1, comp1, root1, bias1,
           V2, comp2, root2, bias2):
    src = edge_index[0].astype(jnp.int32)
    tgt = edge_index[1].astype(jnp.int32)
    nt = node_type.astype(jnp.int32)

    w1, w2 = _weights(src, tgt, nt, comp1.reshape(R), comp2.reshape(R))

    agg1 = _agg(x, src, tgt, w1)
    x1 = _tc(agg1[0], agg1[1], x, V1[0], root1, bias1.reshape(1, D))
    agg2 = _agg(x1, src, tgt, w2)
    x2 = _tc(agg2[0], agg2[1], x1, V2[0], root2, bias2.reshape(1, D))
    return jnp.concatenate([x1, x2], axis=1)


# speedup vs baseline: 14.1689x; 14.1689x over previous
import functools

import jax
import jax.numpy as jnp
from jax import lax
from jax.experimental import pallas as pl
from jax.experimental.pallas import tpu as pltpu
from jax.experimental.pallas import tpu_sc as plsc

N = 10000
E = 320000
D = 128
T = 4
R = 16

NC = 2
NS = 16
NW = NC * NS
L = 16

NP = 10240
CH = 128
NCH = E // CH
LANE_IT = CH // L


def _chunk_range(worker, num_workers):
    q = NCH // num_workers
    r = NCH % num_workers
    return worker * q + jnp.minimum(worker, r)


def _weights_body(src_hbm, tgt_hbm, nt_hbm, c1_hbm, c2_hbm, w1_hbm, w2_hbm,
                  nt_v, srcv, tgtv, segv, relv, onesv, cntv, w1v, w2v,
                  c1v, c2v, zbuf, cnt_sh, gsem):
    cid = lax.axis_index("c")
    sid = lax.axis_index("s")
    wid = cid * NS + sid

    pltpu.sync_copy(nt_hbm, nt_v)
    pltpu.sync_copy(c1_hbm, c1v)
    pltpu.sync_copy(c2_hbm, c2v)

    zeros16 = jnp.zeros((L,), jnp.float32)
    ones16 = jnp.ones((L,), jnp.float32)

    def init_bufs(i, carry):
        zbuf[pl.ds(i * L, L)] = zeros16
        return carry
    lax.fori_loop(0, (N * R // NS) // L, init_bufs, 0)
    for j in range(LANE_IT):
        onesv[pl.ds(j * L, L)] = ones16

    pltpu.sync_copy(zbuf, cnt_sh.at[pl.ds(sid * (N * R // NS), N * R // NS)])
    plsc.subcore_barrier()

    def compute_seg(base):
        pltpu.sync_copy(src_hbm.at[pl.ds(base, CH)], srcv)
        pltpu.sync_copy(tgt_hbm.at[pl.ds(base, CH)], tgtv)
        for k in range(LANE_IT):
            s16 = srcv[pl.ds(k * L, L)]
            t16 = tgtv[pl.ds(k * L, L)]
            ts = plsc.load_gather(nt_v, [s16])
            tt = plsc.load_gather(nt_v, [t16])
            rel = tt * T + ts
            segv[pl.ds(k * L, L)] = t16 * R + rel
            relv[pl.ds(k * L, L)] = rel

    c0 = _chunk_range(sid, NS)
    cn = NCH // NS + (sid < NCH % NS).astype(jnp.int32)

    def count_chunk(ci, carry):
        compute_seg((c0 + ci) * CH)
        pltpu.sync_copy(onesv, cnt_sh.at[segv], add=True)
        return carry
    lax.fori_loop(0, cn, count_chunk, 0)
    plsc.subcore_barrier()

    w0 = _chunk_range(wid, NW)
    wn = NCH // NW + (wid < NCH % NW).astype(jnp.int32)

    def w_chunk(ci, carry):
        base = (w0 + ci) * CH
        compute_seg(base)
        pltpu.async_copy(cnt_sh.at[segv], cntv, gsem).wait()
        for k in range(LANE_IT):
            c16 = cntv[pl.ds(k * L, L)]
            inv = 1.0 / jnp.maximum(c16, 1.0)
            rel16 = relv[pl.ds(k * L, L)]
            w1v[pl.ds(k * L, L)] = plsc.load_gather(c1v, [rel16]) * inv
            w2v[pl.ds(k * L, L)] = plsc.load_gather(c2v, [rel16]) * inv
        pltpu.sync_copy(w1v, w1_hbm.at[pl.ds(base, CH)])
        pltpu.sync_copy(w2v, w2_hbm.at[pl.ds(base, CH)])
        return carry
    lax.fori_loop(0, wn, w_chunk, 0)


def _weights(src, tgt, node_type, comp1, comp2):
    mesh = plsc.VectorSubcoreMesh(core_axis_name="c", subcore_axis_name="s", num_cores=NC, num_subcores=NS)
    f = pl.kernel(
        _weights_body,
        out_type=(jax.ShapeDtypeStruct((E,), jnp.float32),
                  jax.ShapeDtypeStruct((E,), jnp.float32)),
        mesh=mesh,
        scratch_types=[
            pltpu.VMEM((N,), jnp.int32),
            pltpu.VMEM((CH,), jnp.int32),
            pltpu.VMEM((CH,), jnp.int32),
            pltpu.VMEM((CH,), jnp.int32),
            pltpu.VMEM((CH,), jnp.int32),
            pltpu.VMEM((CH,), jnp.float32),
            pltpu.VMEM((CH,), jnp.float32),
            pltpu.VMEM((CH,), jnp.float32),
            pltpu.VMEM((CH,), jnp.float32),
            pltpu.VMEM((R,), jnp.float32),
            pltpu.VMEM((R,), jnp.float32),
            pltpu.VMEM((N * R // NS,), jnp.float32),
            pltpu.VMEM_SHARED((N * R,), jnp.float32),
            pltpu.SemaphoreType.DMA,
        ],
        name="rgcn_weights",
        compiler_params=pltpu.CompilerParams(needs_layout_passes=False),
    )
    return f(src, tgt, node_type, comp1, comp2)


def _agg_body(x_hbm, src_hbm, tgt_hbm, w_hbm, out_hbm,
              srcv, tgtv, wv, rows, agg_sh, gsem):
    cid = lax.axis_index("c")
    sid = lax.axis_index("s")
    wid = cid * NS + sid

    zeros16 = jnp.zeros((L,), jnp.float32)

    def zrow(i, carry):
        for j in range(D // L):
            rows[i, pl.ds(j * L, L)] = zeros16
        return carry
    lax.fori_loop(0, CH, zrow, 0)

    rows_per_tile = NP // NS
    r0 = sid * rows_per_tile
    nrow = rows_per_tile // 5
    for i in range(5):
        pltpu.sync_copy(rows.at[pl.ds(0, nrow)],
                        agg_sh.at[pl.ds(r0 + i * nrow, nrow)])
    plsc.subcore_barrier()

    c0 = _chunk_range(wid, NW)
    cn = NCH // NW + (wid < NCH % NW).astype(jnp.int32)

    def chunk(ci, carry):
        base = (c0 + ci) * CH
        pltpu.sync_copy(src_hbm.at[pl.ds(base, CH)], srcv)
        pltpu.sync_copy(tgt_hbm.at[pl.ds(base, CH)], tgtv)
        pltpu.sync_copy(w_hbm.at[pl.ds(base, CH)], wv)
        pltpu.async_copy(x_hbm.at[srcv], rows, gsem).wait()

        def scale(i, icarry):
            w16 = plsc.load_gather(wv, [jnp.full((L,), i, jnp.int32)])
            for j in range(D // L):
                rows[i, pl.ds(j * L, L)] = rows[i, pl.ds(j * L, L)] * w16
            return icarry
        lax.fori_loop(0, CH, scale, 0)

        pltpu.sync_copy(rows, agg_sh.at[tgtv], add=True)
        return carry
    lax.fori_loop(0, cn, chunk, 0)
    plsc.subcore_barrier()

    for i in range(5):
        pltpu.sync_copy(agg_sh.at[pl.ds(r0 + i * nrow, nrow)],
                        out_hbm.at[cid, pl.ds(r0 + i * nrow, nrow)])


def _agg(x, src, tgt, w):
    mesh = plsc.VectorSubcoreMesh(core_axis_name="c", subcore_axis_name="s", num_cores=NC, num_subcores=NS)
    f = pl.kernel(
        _agg_body,
        out_type=jax.ShapeDtypeStruct((NC, NP, D), jnp.float32),
        mesh=mesh,
        scratch_types=[
            pltpu.VMEM((CH,), jnp.int32),
            pltpu.VMEM((CH,), jnp.int32),
            pltpu.VMEM((CH,), jnp.float32),
            pltpu.VMEM((CH, D), jnp.float32),
            pltpu.VMEM_SHARED((NP, D), jnp.float32),
            pltpu.SemaphoreType.DMA,
        ],
        name="rgcn_agg",
        compiler_params=pltpu.CompilerParams(needs_layout_passes=False),
    )
    return f(x, src, tgt, w)


_BR = 1000


def _tc_body(a0_ref, a1_ref, x_ref, v_ref, root_ref, b_ref, o_ref):
    agg = a0_ref[...] + a1_ref[...]
    acc = jnp.dot(agg, v_ref[...], preferred_element_type=jnp.float32)
    acc += jnp.dot(x_ref[...], root_ref[...], preferred_element_type=jnp.float32)
    o_ref[...] = jax.nn.sigmoid(acc + b_ref[...])


def _tc(agg0, agg1, x, V, root, bias):
    grid = (N // _BR,)
    row_spec = pl.BlockSpec((_BR, D), lambda i: (i, 0))
    mat_spec = pl.BlockSpec((D, D), lambda i: (0, 0))
    return pl.pallas_call(
        _tc_body,
        grid=grid,
        in_specs=[row_spec, row_spec, row_spec, mat_spec, mat_spec,
                  pl.BlockSpec((1, D), lambda i: (0, 0))],
        out_specs=row_spec,
        out_shape=jax.ShapeDtypeStruct((N, D), jnp.float32),
    )(agg0, agg1, x, V, root, bias)


def kernel(x, edge_index, node_type, V1, comp1, root1, bias1,
           V2, comp2, root2, bias2):
    src = edge_index[0].astype(jnp.int32)
    tgt = edge_index[1].astype(jnp.int32)
    nt = node_type.astype(jnp.int32)

    w1, w2 = _weights(src, tgt, nt, comp1.reshape(R), comp2.reshape(R))

    agg1 = _agg(x, src, tgt, w1)
    x1 = _tc(agg1[0], agg1[1], x, V1[0], root1, bias1.reshape(1, D))
    agg2 = _agg(x1, src, tgt, w2)
    x2 = _tc(agg2[0], agg2[1], x1, V2[0], root2, bias2.reshape(1, D))
    return jnp.concatenate([x1, x2], axis=1)


# speedup vs baseline: 22.7056x; 1.6025x over previous
import functools

import jax
import jax.numpy as jnp
from jax import lax
from jax.experimental import pallas as pl
from jax.experimental.pallas import tpu as pltpu
from jax.experimental.pallas import tpu_sc as plsc

N = 10000
E = 320000
D = 128
T = 4
R = 16

NC = 2
NS = 16
NW = NC * NS
L = 16

NP = 10240
CH = 128
NCH = E // CH
LANE_IT = CH // L
CPT = NCH // NW
REM = NCH % NW


def _chunk_range(worker, num_workers):
    q = NCH // num_workers
    r = NCH % num_workers
    return worker * q + jnp.minimum(worker, r)


def _weights_body(src_hbm, tgt_hbm, nt_hbm, c1_hbm, c2_hbm, eb_hbm,
                  nt_v, srcv, tgtv, segv, relv, onesv, cntv, obuf,
                  c1v, c2v, zbuf, cnt_sh, gsem):
    cid = lax.axis_index("c")
    sid = lax.axis_index("s")
    wid = cid * NS + sid

    pltpu.sync_copy(nt_hbm, nt_v)
    pltpu.sync_copy(c1_hbm, c1v)
    pltpu.sync_copy(c2_hbm, c2v)

    zeros16 = jnp.zeros((L,), jnp.float32)
    ones16 = jnp.ones((L,), jnp.float32)

    def init_bufs(i, carry):
        zbuf[pl.ds(i * L, L)] = zeros16
        return carry
    lax.fori_loop(0, (N * R // NS) // L, init_bufs, 0)
    for j in range(LANE_IT):
        onesv[pl.ds(j * L, L)] = ones16

    pltpu.sync_copy(zbuf, cnt_sh.at[pl.ds(sid * (N * R // NS), N * R // NS)])
    plsc.subcore_barrier()

    def compute_seg(base):
        pltpu.sync_copy(src_hbm.at[pl.ds(base, CH)], srcv)
        pltpu.sync_copy(tgt_hbm.at[pl.ds(base, CH)], tgtv)
        for k in range(LANE_IT):
            s16 = srcv[pl.ds(k * L, L)]
            t16 = tgtv[pl.ds(k * L, L)]
            ts = plsc.load_gather(nt_v, [s16])
            tt = plsc.load_gather(nt_v, [t16])
            rel = tt * T + ts
            segv[pl.ds(k * L, L)] = t16 * R + rel
            relv[pl.ds(k * L, L)] = rel

    c0 = _chunk_range(sid, NS)
    cn = NCH // NS + (sid < NCH % NS).astype(jnp.int32)

    def count_chunk(ci, carry):
        compute_seg((c0 + ci) * CH)
        pltpu.sync_copy(onesv, cnt_sh.at[segv], add=True)
        return carry
    lax.fori_loop(0, cn, count_chunk, 0)
    plsc.subcore_barrier()

    w0 = _chunk_range(wid, NW)
    wn = NCH // NW + (wid < NCH % NW).astype(jnp.int32)

    def w_chunk(ci, carry):
        cidx = w0 + ci
        compute_seg(cidx * CH)
        pltpu.async_copy(cnt_sh.at[segv], cntv, gsem).wait()
        for k in range(LANE_IT):
            sl = pl.ds(k * L, L)
            c16 = cntv[sl]
            inv = 1.0 / jnp.maximum(c16, 1.0)
            rel16 = relv[sl]
            obuf[0, sl] = srcv[sl]
            obuf[1, sl] = tgtv[sl]
            obuf[2, sl] = plsc.bitcast(plsc.load_gather(c1v, [rel16]) * inv,
                                       jnp.int32)
            obuf[3, sl] = plsc.bitcast(plsc.load_gather(c2v, [rel16]) * inv,
                                       jnp.int32)
        pltpu.sync_copy(obuf, eb_hbm.at[cidx])
        return carry
    lax.fori_loop(0, wn, w_chunk, 0)


def _weights(src, tgt, node_type, comp1, comp2):
    mesh = plsc.VectorSubcoreMesh(core_axis_name="c", subcore_axis_name="s",
                                  num_cores=NC, num_subcores=NS)
    f = pl.kernel(
        _weights_body,
        out_type=jax.ShapeDtypeStruct((NCH, 4, CH), jnp.int32),
        mesh=mesh,
        scratch_types=[
            pltpu.VMEM((N,), jnp.int32),
            pltpu.VMEM((CH,), jnp.int32),
            pltpu.VMEM((CH,), jnp.int32),
            pltpu.VMEM((CH,), jnp.int32),
            pltpu.VMEM((CH,), jnp.int32),
            pltpu.VMEM((CH,), jnp.float32),
            pltpu.VMEM((CH,), jnp.float32),
            pltpu.VMEM((4, CH), jnp.int32),
            pltpu.VMEM((R,), jnp.float32),
            pltpu.VMEM((R,), jnp.float32),
            pltpu.VMEM((N * R // NS,), jnp.float32),
            pltpu.VMEM_SHARED((N * R,), jnp.float32),
            pltpu.SemaphoreType.DMA,
        ],
        name="rgcn_weights",
        compiler_params=pltpu.CompilerParams(needs_layout_passes=False),
    )
    return f(src, tgt, node_type, comp1, comp2)


def _make_agg_body(wrow):
    def _agg_body(x_hbm, eb_hbm, out_hbm,
                  ibuf, rows, tgtv, agg_sh, d0, d1, g0, g1, s0, s1):
        cid = lax.axis_index("c")
        sid = lax.axis_index("s")
        wid = cid * NS + sid
        dsem = (d0, d1)
        gsem = (g0, g1)
        ssem = (s0, s1)

        def idx_desc(ci, b):
            return pltpu.make_async_copy(eb_hbm.at[ci], ibuf.at[b], dsem[b])

        def gather_desc(b):
            return pltpu.make_async_copy(x_hbm.at[ibuf.at[b, 0]],
                                         rows.at[b], gsem[b])

        def scat_desc(b):
            return pltpu.make_async_copy(rows.at[b],
                                         agg_sh.at[tgtv.at[b]], ssem[b])

        zeros16 = jnp.zeros((L,), jnp.float32)

        def zrow(i, carry):
            for j in range(D // L):
                rows[0, i, pl.ds(j * L, L)] = zeros16
            return carry
        lax.fori_loop(0, CH, zrow, 0)
        rows_per_tile = NP // NS
        r0 = sid * rows_per_tile
        nrow = rows_per_tile // 5
        for i in range(5):
            pltpu.sync_copy(rows.at[0, pl.ds(0, nrow)],
                            agg_sh.at[pl.ds(r0 + i * nrow, nrow)])
        plsc.subcore_barrier()

        base = wid * CPT

        def scale(b):
            for k in range(LANE_IT):
                sl = pl.ds(k * L, L)
                tgtv[b, sl] = ibuf[b, 1, sl]

            def srow(i, carry):
                wbits = plsc.load_gather(ibuf.at[b, wrow],
                                         [jnp.full((L,), i, jnp.int32)])
                w16 = plsc.bitcast(wbits, jnp.float32)
                for j in range(D // L):
                    sl = pl.ds(j * L, L)
                    rows[b, i, sl] = rows[b, i, sl] * w16
                return carry
            lax.fori_loop(0, CH, srow, 0)

        idx_desc(base, 0).start()
        idx_desc(base + 1, 1).start()
        idx_desc(base, 0).wait()
        gather_desc(0).start()

        def step(i, b):
            nb = 1 - b
            gather_desc(b).wait()

            @pl.when(i <= CPT - 2)
            def _():
                idx_desc(base + i + 1, nb).wait()

                @pl.when(i >= 1)
                def _():
                    scat_desc(nb).wait()
                gather_desc(nb).start()

            scale(b)
            pltpu.async_copy(rows.at[b], agg_sh.at[tgtv.at[b]], ssem[b],
                             add=True)

            @pl.when(i <= CPT - 3)
            def _():
                idx_desc(base + i + 2, b).start()

        def pair(p, carry):
            step(p * 2, 0)
            step(p * 2 + 1, 1)
            return carry
        lax.fori_loop(0, CPT // 2, pair, 0)

        scat_desc(0).wait()
        scat_desc(1).wait()

        @pl.when(wid < REM)
        def _():
            ci = NW * CPT + wid
            pltpu.sync_copy(eb_hbm.at[ci], ibuf.at[0])
            pltpu.async_copy(x_hbm.at[ibuf.at[0, 0]], rows.at[0], g0).wait()
            scale(0)
            pltpu.async_copy(rows.at[0], agg_sh.at[tgtv.at[0]], s0,
                             add=True).wait()

        plsc.subcore_barrier()
        for i in range(5):
            pltpu.sync_copy(agg_sh.at[pl.ds(r0 + i * nrow, nrow)],
                            out_hbm.at[cid, pl.ds(r0 + i * nrow, nrow)])
    return _agg_body


def _agg(x, eb, wrow):
    mesh = plsc.VectorSubcoreMesh(core_axis_name="c", subcore_axis_name="s",
                                  num_cores=NC, num_subcores=NS)
    f = pl.kernel(
        _make_agg_body(wrow),
        out_type=jax.ShapeDtypeStruct((NC, NP, D), jnp.float32),
        mesh=mesh,
        scratch_types=[
            pltpu.VMEM((2, 4, CH), jnp.int32),
            pltpu.VMEM((2, CH, D), jnp.float32),
            pltpu.VMEM((2, CH), jnp.int32),
            pltpu.VMEM_SHARED((NP, D), jnp.float32),
            pltpu.SemaphoreType.DMA,
            pltpu.SemaphoreType.DMA,
            pltpu.SemaphoreType.DMA,
            pltpu.SemaphoreType.DMA,
            pltpu.SemaphoreType.DMA,
            pltpu.SemaphoreType.DMA,
        ],
        name="rgcn_agg",
        compiler_params=pltpu.CompilerParams(needs_layout_passes=False),
    )
    return f(x, eb)


_BR = 1000


def _tc_body(a0_ref, a1_ref, x_ref, v_ref, root_ref, b_ref, o_ref):
    agg = a0_ref[...] + a1_ref[...]
    acc = jnp.dot(agg, v_ref[...], preferred_element_type=jnp.float32)
    acc += jnp.dot(x_ref[...], root_ref[...], preferred_element_type=jnp.float32)
    o_ref[...] = jax.nn.sigmoid(acc + b_ref[...])


def _tc(agg0, agg1, x, V, root, bias):
    grid = (N // _BR,)
    row_spec = pl.BlockSpec((_BR, D), lambda i: (i, 0))
    mat_spec = pl.BlockSpec((D, D), lambda i: (0, 0))
    return pl.pallas_call(
        _tc_body,
        grid=grid,
        in_specs=[row_spec, row_spec, row_spec, mat_spec, mat_spec,
                  pl.BlockSpec((1, D), lambda i: (0, 0))],
        out_specs=row_spec,
        out_shape=jax.ShapeDtypeStruct((N, D), jnp.float32),
    )(agg0, agg1, x, V, root, bias)


def kernel(x, edge_index, node_type, V1, comp1, root1, bias1,
           V2, comp2, root2, bias2):
    src = edge_index[0].astype(jnp.int32)
    tgt = edge_index[1].astype(jnp.int32)
    nt = node_type.astype(jnp.int32)

    eb = _weights(src, tgt, nt, comp1.reshape(R), comp2.reshape(R))

    agg1 = _agg(x, eb, 2)
    x1 = _tc(agg1[0], agg1[1], x, V1[0], root1, bias1.reshape(1, D))
    agg2 = _agg(x1, eb, 3)
    x2 = _tc(agg2[0], agg2[1], x1, V2[0], root2, bias2.reshape(1, D))
    return jnp.concatenate([x1, x2], axis=1)


# speedup vs baseline: 38.4668x; 1.6942x over previous
import jax
import jax.numpy as jnp
from jax import lax
from jax.experimental import pallas as pl
from jax.experimental.pallas import tpu as pltpu
from jax.experimental.pallas import tpu_sc as plsc

N = 10000
E = 320000
D = 128
T = 4
R = 16

NC = 2
NS = 16
NW = NC * NS
L = 16

NP = 10240
CH = 128
NCH = E // CH
LANE_IT = CH // L
CPT = NCH // NW
REM = NCH % NW
CPS = NCH // NS
REMS = NCH % NS


def _weights_body(st_hbm, nt_hbm, c1_hbm, c2_hbm, eb_hbm,
                  nt_v, stbuf, segb, relb, onesv, cntb, obuf,
                  c1v, c2v, zbuf,
                  cnt_sh, sd0, sd1, sc0, sc1, cg0, cg1, so0, so1):
    cid = lax.axis_index("c")
    sid = lax.axis_index("s")
    wid = cid * NS + sid
    sdsem = (sd0, sd1)
    scsem = (sc0, sc1)
    cgsem = (cg0, cg1)
    sosem = (so0, so1)

    pltpu.sync_copy(nt_hbm, nt_v)
    pltpu.sync_copy(c1_hbm, c1v)
    pltpu.sync_copy(c2_hbm, c2v)

    zeros16 = jnp.zeros((L,), jnp.float32)
    ones16 = jnp.ones((L,), jnp.float32)

    def init_bufs(i, carry):
        zbuf[pl.ds(i * L, L)] = zeros16
        return carry
    lax.fori_loop(0, (N * R // NS) // L, init_bufs, 0)
    for j in range(LANE_IT):
        onesv[pl.ds(j * L, L)] = ones16

    pltpu.sync_copy(zbuf, cnt_sh.at[pl.ds(sid * (N * R // NS), N * R // NS)])
    plsc.subcore_barrier()

    def st_desc(ci, b):
        return pltpu.make_async_copy(st_hbm.at[ci], stbuf.at[b], sdsem[b])

    def seg_compute(b, with_obuf):
        for k in range(LANE_IT):
            sl = pl.ds(k * L, L)
            s16 = stbuf[b, 0, sl]
            t16 = stbuf[b, 1, sl]
            ts = plsc.load_gather(nt_v, [s16])
            tt = plsc.load_gather(nt_v, [t16])
            rel = tt * T + ts
            segb[b, sl] = t16 * R + rel
            if with_obuf:
                relb[b, sl] = rel
                obuf[b, 0, sl] = s16
                obuf[b, 1, sl] = t16

    cbase = sid * CPS

    def count_step(i, b):
        st_desc(cbase + i, b).wait()

        @pl.when(i >= 2)
        def _():
            pltpu.make_async_copy(onesv, cnt_sh.at[segb.at[b]],
                                  scsem[b]).wait()
        seg_compute(b, False)

        @pl.when(i <= CPS - 3)
        def _():
            st_desc(cbase + i + 2, b).start()
        pltpu.async_copy(onesv, cnt_sh.at[segb.at[b]], scsem[b], add=True)

    st_desc(cbase, 0).start()
    st_desc(cbase + 1, 1).start()

    def count_pair(p, carry):
        count_step(p * 2, 0)
        count_step(p * 2 + 1, 1)
        return carry
    lax.fori_loop(0, CPS // 2, count_pair, 0)
    pltpu.make_async_copy(onesv, cnt_sh.at[segb.at[0]], scsem[0]).wait()
    pltpu.make_async_copy(onesv, cnt_sh.at[segb.at[1]], scsem[1]).wait()

    @pl.when(sid < REMS)
    def _():
        ci = NS * CPS + sid
        pltpu.sync_copy(st_hbm.at[ci], stbuf.at[0])
        seg_compute(0, False)
        pltpu.async_copy(onesv, cnt_sh.at[segb.at[0]], scsem[0],
                         add=True).wait()

    plsc.subcore_barrier()

    wbase = wid * CPT

    def w_finish(i_prev, nb):
        pltpu.make_async_copy(cnt_sh.at[segb.at[nb]], cntb.at[nb],
                              cgsem[nb]).wait()
        for k in range(LANE_IT):
            sl = pl.ds(k * L, L)
            inv = 1.0 / jnp.maximum(cntb[nb, sl], 1.0)
            rel16 = relb[nb, sl]
            obuf[nb, 2, sl] = plsc.bitcast(
                plsc.load_gather(c1v, [rel16]) * inv, jnp.int32)
            obuf[nb, 3, sl] = plsc.bitcast(
                plsc.load_gather(c2v, [rel16]) * inv, jnp.int32)
        pltpu.async_copy(obuf.at[nb], eb_hbm.at[wbase + i_prev], sosem[nb])

    def w_step(i, b):
        nb = 1 - b
        st_desc(wbase + i, b).wait()

        @pl.when(i >= 2)
        def _():
            pltpu.make_async_copy(obuf.at[b], eb_hbm.at[wbase + i - 2],
                                  sosem[b]).wait()
        seg_compute(b, True)

        @pl.when(i <= CPT - 3)
        def _():
            st_desc(wbase + i + 2, b).start()
        pltpu.async_copy(cnt_sh.at[segb.at[b]], cntb.at[b], cgsem[b])

        @pl.when(i >= 1)
        def _():
            w_finish(i - 1, nb)

    st_desc(wbase, 0).start()
    st_desc(wbase + 1, 1).start()

    def w_pair(p, carry):
        w_step(p * 2, 0)
        w_step(p * 2 + 1, 1)
        return carry
    lax.fori_loop(0, CPT // 2, w_pair, 0)
    w_finish(CPT - 1, 1)
    pltpu.make_async_copy(obuf.at[0], eb_hbm.at[wbase + CPT - 2],
                          sosem[0]).wait()
    pltpu.make_async_copy(obuf.at[1], eb_hbm.at[wbase + CPT - 1],
                          sosem[1]).wait()

    @pl.when(wid < REM)
    def _():
        ci = NW * CPT + wid
        pltpu.sync_copy(st_hbm.at[ci], stbuf.at[0])
        seg_compute(0, True)
        pltpu.async_copy(cnt_sh.at[segb.at[0]], cntb.at[0], cgsem[0]).wait()
        for k in range(LANE_IT):
            sl = pl.ds(k * L, L)
            inv = 1.0 / jnp.maximum(cntb[0, sl], 1.0)
            rel16 = relb[0, sl]
            obuf[0, 2, sl] = plsc.bitcast(
                plsc.load_gather(c1v, [rel16]) * inv, jnp.int32)
            obuf[0, 3, sl] = plsc.bitcast(
                plsc.load_gather(c2v, [rel16]) * inv, jnp.int32)
        pltpu.sync_copy(obuf.at[0], eb_hbm.at[ci])


def _weights(st, node_type, comp1, comp2):
    mesh = plsc.VectorSubcoreMesh(core_axis_name="c", subcore_axis_name="s",
                                  num_cores=NC, num_subcores=NS)
    f = pl.kernel(
        _weights_body,
        out_type=jax.ShapeDtypeStruct((NCH, 4, CH), jnp.int32),
        mesh=mesh,
        scratch_types=[
            pltpu.VMEM((N,), jnp.int32),
            pltpu.VMEM((2, 2, CH), jnp.int32),
            pltpu.VMEM((2, CH), jnp.int32),
            pltpu.VMEM((2, CH), jnp.int32),
            pltpu.VMEM((CH,), jnp.float32),
            pltpu.VMEM((2, CH), jnp.float32),
            pltpu.VMEM((2, 4, CH), jnp.int32),
            pltpu.VMEM((R,), jnp.float32),
            pltpu.VMEM((R,), jnp.float32),
            pltpu.VMEM((N * R // NS,), jnp.float32),
            pltpu.VMEM_SHARED((N * R,), jnp.float32),
            pltpu.SemaphoreType.DMA,
            pltpu.SemaphoreType.DMA,
            pltpu.SemaphoreType.DMA,
            pltpu.SemaphoreType.DMA,
            pltpu.SemaphoreType.DMA,
            pltpu.SemaphoreType.DMA,
            pltpu.SemaphoreType.DMA,
            pltpu.SemaphoreType.DMA,
        ],
        name="rgcn_weights",
        compiler_params=pltpu.CompilerParams(needs_layout_passes=False),
    )
    return f(st, node_type, comp1, comp2)


def _make_agg_body(wrow):
    def _agg_body(x_hbm, eb_hbm, out_hbm,
                  ibuf, rows, tgtv, agg_sh, d0, d1, g0, g1, s0, s1):
        cid = lax.axis_index("c")
        sid = lax.axis_index("s")
        wid = cid * NS + sid
        dsem = (d0, d1)
        gsem = (g0, g1)
        ssem = (s0, s1)

        def idx_desc(ci, b):
            return pltpu.make_async_copy(eb_hbm.at[ci], ibuf.at[b], dsem[b])

        def gather_desc(b):
            return pltpu.make_async_copy(x_hbm.at[ibuf.at[b, 0]],
                                         rows.at[b], gsem[b])

        def scat_desc(b):
            return pltpu.make_async_copy(rows.at[b],
                                         agg_sh.at[tgtv.at[b]], ssem[b])

        zeros16 = jnp.zeros((L,), jnp.float32)

        def zrow(i, carry):
            for j in range(D // L):
                rows[0, i, pl.ds(j * L, L)] = zeros16
            return carry
        lax.fori_loop(0, CH, zrow, 0)
        rows_per_tile = NP // NS
        r0 = sid * rows_per_tile
        nrow = rows_per_tile // 5
        for i in range(5):
            pltpu.sync_copy(rows.at[0, pl.ds(0, nrow)],
                            agg_sh.at[pl.ds(r0 + i * nrow, nrow)])
        plsc.subcore_barrier()

        base = wid * CPT

        def scale(b):
            for k in range(LANE_IT):
                sl = pl.ds(k * L, L)
                tgtv[b, sl] = ibuf[b, 1, sl]

            @plsc.parallel_loop(0, CH, 1, unroll=4)
            def srow(i):
                wbits = plsc.load_gather(ibuf.at[b, wrow],
                                         [jnp.full((L,), i, jnp.int32)])
                w16 = plsc.bitcast(wbits, jnp.float32)
                for j in range(D // L):
                    sl = pl.ds(j * L, L)
                    rows[b, i, sl] = rows[b, i, sl] * w16

        idx_desc(base, 0).start()
        idx_desc(base + 1, 1).start()
        idx_desc(base, 0).wait()
        gather_desc(0).start()

        def step(i, b):
            nb = 1 - b
            gather_desc(b).wait()

            @pl.when(i <= CPT - 2)
            def _():
                idx_desc(base + i + 1, nb).wait()

                @pl.when(i >= 1)
                def _():
                    scat_desc(nb).wait()
                gather_desc(nb).start()

            scale(b)
            pltpu.async_copy(rows.at[b], agg_sh.at[tgtv.at[b]], ssem[b],
                             add=True)

            @pl.when(i <= CPT - 3)
            def _():
                idx_desc(base + i + 2, b).start()

        def pair(p, carry):
            step(p * 2, 0)
            step(p * 2 + 1, 1)
            return carry
        lax.fori_loop(0, CPT // 2, pair, 0)

        scat_desc(0).wait()
        scat_desc(1).wait()

        @pl.when(wid < REM)
        def _():
            ci = NW * CPT + wid
            pltpu.sync_copy(eb_hbm.at[ci], ibuf.at[0])
            pltpu.async_copy(x_hbm.at[ibuf.at[0, 0]], rows.at[0], g0).wait()
            scale(0)
            pltpu.async_copy(rows.at[0], agg_sh.at[tgtv.at[0]], s0,
                             add=True).wait()

        plsc.subcore_barrier()
        for i in range(5):
            pltpu.sync_copy(agg_sh.at[pl.ds(r0 + i * nrow, nrow)],
                            out_hbm.at[cid, pl.ds(r0 + i * nrow, nrow)])
    return _agg_body


def _agg(x, eb, wrow):
    mesh = plsc.VectorSubcoreMesh(core_axis_name="c", subcore_axis_name="s",
                                  num_cores=NC, num_subcores=NS)
    f = pl.kernel(
        _make_agg_body(wrow),
        out_type=jax.ShapeDtypeStruct((NC, NP, D), jnp.float32),
        mesh=mesh,
        scratch_types=[
            pltpu.VMEM((2, 4, CH), jnp.int32),
            pltpu.VMEM((2, CH, D), jnp.float32),
            pltpu.VMEM((2, CH), jnp.int32),
            pltpu.VMEM_SHARED((NP, D), jnp.float32),
            pltpu.SemaphoreType.DMA,
            pltpu.SemaphoreType.DMA,
            pltpu.SemaphoreType.DMA,
            pltpu.SemaphoreType.DMA,
            pltpu.SemaphoreType.DMA,
            pltpu.SemaphoreType.DMA,
        ],
        name="rgcn_agg",
        compiler_params=pltpu.CompilerParams(needs_layout_passes=False),
    )
    return f(x, eb)


_BR = 1000


def _tc_body(a0_ref, a1_ref, x_ref, v_ref, root_ref, b_ref, o_ref):
    agg = a0_ref[...] + a1_ref[...]
    acc = jnp.dot(agg, v_ref[...], preferred_element_type=jnp.float32)
    acc += jnp.dot(x_ref[...], root_ref[...], preferred_element_type=jnp.float32)
    o_ref[...] = jax.nn.sigmoid(acc + b_ref[...])


def _tc(agg0, agg1, x, V, root, bias):
    grid = (N // _BR,)
    row_spec = pl.BlockSpec((_BR, D), lambda i: (i, 0))
    mat_spec = pl.BlockSpec((D, D), lambda i: (0, 0))
    return pl.pallas_call(
        _tc_body,
        grid=grid,
        in_specs=[row_spec, row_spec, row_spec, mat_spec, mat_spec,
                  pl.BlockSpec((1, D), lambda i: (0, 0))],
        out_specs=row_spec,
        out_shape=jax.ShapeDtypeStruct((N, D), jnp.float32),
    )(agg0, agg1, x, V, root, bias)


def kernel(x, edge_index, node_type, V1, comp1, root1, bias1,
           V2, comp2, root2, bias2):
    ei = edge_index.astype(jnp.int32)
    nt = node_type.astype(jnp.int32)
    st = jnp.transpose(ei.reshape(2, NCH, CH), (1, 0, 2))

    eb = _weights(st, nt, comp1.reshape(R), comp2.reshape(R))

    agg1 = _agg(x, eb, 2)
    x1 = _tc(agg1[0], agg1[1], x, V1[0], root1, bias1.reshape(1, D))
    agg2 = _agg(x1, eb, 3)
    x2 = _tc(agg2[0], agg2[1], x1, V2[0], root2, bias2.reshape(1, D))
    return jnp.concatenate([x1, x2], axis=1)
